# baseline (speedup 1.0000x reference)
import dataclasses
import functools

import jax
import jax.numpy as jnp
from jax import lax
from jax.experimental import pallas as pl
from jax.experimental.pallas import tpu as pltpu
from jax.experimental.pallas import tpu_sc as plsc

NC = 2
NS = 16
NW = NC * NS
K = 128
ZR = 64
D = 128


def _round_up(v, m):
    return (v + m - 1) // m * m


F0 = 0.64


@functools.lru_cache(maxsize=None)
def _make_agg(n_rows, ep):
    e0 = (int(ep * F0) // (NS * K * 2)) * (NS * K * 2)
    ept = (e0 // NS, (ep - e0) // NS)
    nch = (ept[0] // K, ept[1] // K)
    rpt = n_rows // NS
    mesh = plsc.VectorSubcoreMesh(
        core_axis_name="c", subcore_axis_name="s", num_cores=NC, num_subcores=NS
    )

    @functools.partial(
        pl.kernel,
        out_type=jax.ShapeDtypeStruct((NC * n_rows, D), jnp.float32),
        mesh=mesh,
        scratch_types=[
            [pltpu.VMEM((K,), jnp.int32) for _ in range(2)],
            [pltpu.VMEM((K,), jnp.int32) for _ in range(2)],
            pltpu.VMEM((K, D), jnp.float32),
            pltpu.VMEM((K, D), jnp.float32),
            pltpu.VMEM_SHARED((n_rows, D), jnp.float32),
            [pltpu.SemaphoreType.DMA for _ in range(2)],
            [pltpu.SemaphoreType.DMA for _ in range(2)],
        ],
    )
    def agg(table_hbm, src_hbm, dst_hbm, zeros_hbm, out_hbm,
            sidx, didx, rows, zbuf, acc, isem, dsem):
        cid = lax.axis_index("c")
        sid = lax.axis_index("s")

        pltpu.sync_copy(zeros_hbm, zbuf)
        nfull = rpt // K
        for z in range(nfull):
            pltpu.sync_copy(zbuf, acc.at[pl.ds(sid * rpt + z * K, K)])
        if rpt % K:
            pltpu.sync_copy(zbuf.at[pl.ds(0, rpt % K)],
                            acc.at[pl.ds(sid * rpt + nfull * K, rpt % K)])
        plsc.subcore_barrier()

        base = jnp.where(cid == 0, sid * ept[0], e0 + sid * ept[1])
        nch_c = jnp.where(cid == 0, nch[0], nch[1])

        pltpu.async_copy(src_hbm.at[pl.ds(base, K)], sidx[0], isem[0])
        pltpu.async_copy(dst_hbm.at[pl.ds(base, K)], didx[0], dsem[0])

        @pl.loop(0, nch_c // 2)
        def _(r):
            for b in range(2):
                ci = r * 2 + b
                pltpu.make_async_copy(
                    src_hbm.at[pl.ds(base, K)], sidx[b], isem[b]).wait()
                pltpu.make_async_copy(
                    dst_hbm.at[pl.ds(base, K)], didx[b], dsem[b]).wait()

                @pl.when(ci + 1 < nch_c)
                def _():
                    off = base + (ci + 1) * K
                    pltpu.async_copy(src_hbm.at[pl.ds(off, K)], sidx[1 - b],
                                     isem[1 - b])
                    pltpu.async_copy(dst_hbm.at[pl.ds(off, K)], didx[1 - b],
                                     dsem[1 - b])

                pltpu.sync_copy(table_hbm.at[sidx[b]], rows)
                pltpu.sync_copy(rows, acc.at[didx[b]], add=True)

        plsc.subcore_barrier()
        pltpu.sync_copy(acc.at[pl.ds(sid * rpt, rpt)],
                        out_hbm.at[pl.ds(cid * n_rows + sid * rpt, rpt)])

    return agg


@functools.lru_cache(maxsize=None)
def _make_deg(n_rows, e):
    ept = e // NW
    nfull = ept // K
    rem = ept - nfull * K
    hrows = n_rows // 128
    ctiles = hrows // 8
    mesh = plsc.VectorSubcoreMesh(
        core_axis_name="c", subcore_axis_name="s", num_cores=NC, num_subcores=NS
    )
    cp = pltpu.CompilerParams()
    if "needs_layout_passes" in pltpu.CompilerParams.__dataclass_fields__:
        cp = dataclasses.replace(cp, needs_layout_passes=False)

    @functools.partial(
        pl.kernel,
        out_type=jax.ShapeDtypeStruct((NC * hrows, 128), jnp.float32),
        mesh=mesh,
        compiler_params=cp,
        scratch_types=[
            pltpu.VMEM((K,), jnp.int32),
            pltpu.VMEM((hrows, 128), jnp.float32),
            pltpu.VMEM((hrows,), jnp.int32),
            pltpu.VMEM_SHARED((hrows, 128), jnp.float32),
        ],
    )
    def deg(dst_hbm, iota_hbm, zeros_hbm, out_hbm, didx, hist, iota_v, acc):
        cid = lax.axis_index("c")
        sid = lax.axis_index("s")
        wid = cid * NS + sid

        pltpu.sync_copy(zeros_hbm, hist)

        @pl.when(sid < ctiles)
        def _():
            pltpu.sync_copy(zeros_hbm.at[pl.ds(sid * 8, 8)],
                            acc.at[pl.ds(sid * 8, 8)])

        pltpu.sync_copy(iota_hbm, iota_v)

        base = wid * ept
        ones16 = jnp.full((16,), 1.0, jnp.float32)

        def count16(j):
            v = didx[pl.ds(j * 16, 16)]
            plsc.addupdate_scatter(
                hist,
                [lax.shift_right_logical(v, 7), lax.bitwise_and(v, 127)],
                ones16,
            )

        @pl.loop(0, nfull)
        def _(ci):
            pltpu.sync_copy(dst_hbm.at[pl.ds(base + ci * K, K)], didx)
            for j in range(K // 16):
                count16(j)

        if rem:
            pltpu.sync_copy(dst_hbm.at[pl.ds(base + nfull * K, rem)],
                            didx.at[pl.ds(0, rem)])
            for j in range(rem // 16):
                count16(j)

        plsc.subcore_barrier()
        pltpu.sync_copy(hist, acc.at[iota_v], add=True)
        plsc.subcore_barrier()

        @pl.when(sid < ctiles)
        def _():
            pltpu.sync_copy(acc.at[pl.ds(sid * 8, 8)],
                            out_hbm.at[pl.ds(cid * hrows + sid * 8, 8)])

    return deg


def _prep_body(n, r, x_ref, d0_ref, d1_ref, s0_ref, nb_ref):
    deg = d0_ref[...] + d1_ref[...] + 1.0
    norm = lax.rsqrt(jnp.maximum(deg, 1.0))
    x = x_ref[...]
    rn = jnp.sqrt(jnp.sum(x * x, axis=1, keepdims=True))
    s0_ref[...] = x / jnp.maximum(rn, 1e-12) * norm
    nb_ref[...] = jnp.broadcast_to(norm, x.shape)


def _layer_body(n, r, p0_ref, p1_ref, s_ref, nb_ref, w_ref, b_ref, out_ref):
    nb = nb_ref[...]
    t = (p0_ref[...] + p1_ref[...] + s_ref[...]) * nb
    u = jnp.dot(t, w_ref[...], preferred_element_type=jnp.float32) + b_ref[...]
    sn = jax.nn.relu(u) * nb
    row = pl.program_id(0) * r + lax.broadcasted_iota(jnp.int32, sn.shape, 0)
    out_ref[...] = jnp.where(row < n, sn, 0.0)


def _layer2_body(n, r, p0_ref, p1_ref, s_ref, nb_ref, w_ref, b_ref, w3_ref,
                 g_ref):
    nb = nb_ref[...]
    t = (p0_ref[...] + p1_ref[...] + s_ref[...]) * nb
    u = jnp.dot(t, w_ref[...], preferred_element_type=jnp.float32) + b_ref[...]
    s3 = jax.nn.relu(u) * nb
    row = pl.program_id(0) * r + lax.broadcasted_iota(jnp.int32, s3.shape, 0)
    s3 = jnp.where(row < n, s3, 0.0)
    g_ref[...] = jnp.dot(s3, w3_ref[...], preferred_element_type=jnp.float32)


def _final_body(q0_ref, q1_ref, g_ref, nb_ref, b3_ref, out_ref):
    nb = nb_ref[:, 0:64]
    acc = q0_ref[:, 0:64] + q1_ref[:, 0:64] + g_ref[:, 0:64]
    out_ref[...] = acc * nb + b3_ref[...]


def kernel(x, edge_index, W0, b0, W1, b1, W2, b2, W3, b3):
    n, d_in = x.shape
    d_out = W3.shape[1]
    e = edge_index.shape[1]
    ep = _round_up(e, NW * K)
    np_rows = _round_up(n + 1, 128)
    nd_rows = _round_up(n + 1, 1024)
    hrows = nd_rows // 128
    r = np_rows // 8
    grid = (8,)

    src = edge_index[0]
    dst = edge_index[1]
    pad = ep - e
    srcp = jnp.concatenate([src, jnp.full((pad,), n, jnp.int32)])
    dstp = jnp.concatenate([dst, jnp.zeros((pad,), jnp.int32)])
    zrow = jnp.zeros((K, D), jnp.float32)
    zhist = jnp.zeros((hrows, 128), jnp.float32)
    iota_h = jnp.arange(hrows, dtype=jnp.int32)
    xp = jnp.pad(x, ((0, np_rows - n), (0, 0)))
    w3p = jnp.pad(W3, ((0, 0), (0, d_in - d_out)))

    agg = _make_agg(np_rows, ep)
    degk = _make_deg(nd_rows, e)

    degp = degk(dst, iota_h, zhist)
    d0 = degp[:hrows].reshape(nd_rows, 1)[:np_rows]
    d1 = degp[hrows:].reshape(nd_rows, 1)[:np_rows]

    row_spec = pl.BlockSpec((r, d_in), lambda i: (i, 0))
    row1_spec = pl.BlockSpec((r, 1), lambda i: (i, 0))
    row64_spec = pl.BlockSpec((r, d_out), lambda i: (i, 0))
    w_spec = pl.BlockSpec((d_in, d_in), lambda i: (0, 0))
    b_spec = pl.BlockSpec((1, d_in), lambda i: (0, 0))
    b3_spec = pl.BlockSpec((1, d_out), lambda i: (0, 0))
    fshape = jax.ShapeDtypeStruct((np_rows, d_in), jnp.float32)

    s0, normbc = pl.pallas_call(
        functools.partial(_prep_body, n, r),
        grid=grid,
        in_specs=[row_spec, row1_spec, row1_spec],
        out_specs=[row_spec, row_spec],
        out_shape=[fshape, fshape],
    )(xp, d0, d1)

    layer = pl.pallas_call(
        functools.partial(_layer_body, n, r),
        grid=grid,
        in_specs=[row_spec, row_spec, row_spec, row_spec, w_spec, b_spec],
        out_specs=row_spec,
        out_shape=fshape,
    )
    b0r, b1r, b2r = (bb.reshape(1, -1) for bb in (b0, b1, b2))
    b3r = b3.reshape(1, -1)

    p = agg(s0, srcp, dstp, zrow)
    s1 = layer(p[:np_rows], p[np_rows:], s0, normbc, W0, b0r)
    p = agg(s1, srcp, dstp, zrow)
    s2 = layer(p[:np_rows], p[np_rows:], s1, normbc, W1, b1r)
    p = agg(s2, srcp, dstp, zrow)
    g = pl.pallas_call(
        functools.partial(_layer2_body, n, r),
        grid=grid,
        in_specs=[row_spec, row_spec, row_spec, row_spec, w_spec, b_spec,
                  w_spec],
        out_specs=row_spec,
        out_shape=fshape,
    )(p[:np_rows], p[np_rows:], s2, normbc, W2, b2r, w3p)

    q = agg(g, srcp, dstp, zrow)
    out = pl.pallas_call(
        _final_body,
        grid=grid,
        in_specs=[row_spec, row_spec, row_spec, row_spec, b3_spec],
        out_specs=row64_spec,
        out_shape=jax.ShapeDtypeStruct((np_rows, d_out), jnp.float32),
    )(q[:np_rows], q[np_rows:], g, normbc, b3r)
    return out[:n]


# speedup vs baseline: 1.0569x; 1.0569x over previous
import dataclasses
import functools

import jax
import jax.numpy as jnp
from jax import lax
from jax.experimental import pallas as pl
from jax.experimental.pallas import tpu as pltpu
from jax.experimental.pallas import tpu_sc as plsc

NC = 2
NS = 16
NW = NC * NS
K = 128
ZR = 64
D = 128


def _round_up(v, m):
    return (v + m - 1) // m * m


F0 = 0.68


@functools.lru_cache(maxsize=None)
def _make_agg(n_rows, ep):
    e0 = (int(ep * F0) // (NS * K * 2)) * (NS * K * 2)
    ept = (e0 // NS, (ep - e0) // NS)
    nch = (ept[0] // K, ept[1] // K)
    rpt = n_rows // NS
    mesh = plsc.VectorSubcoreMesh(
        core_axis_name="c", subcore_axis_name="s", num_cores=NC, num_subcores=NS
    )

    @functools.partial(
        pl.kernel,
        out_type=jax.ShapeDtypeStruct((NC * n_rows, D), jnp.float32),
        mesh=mesh,
        scratch_types=[
            [pltpu.VMEM((K,), jnp.int32) for _ in range(2)],
            [pltpu.VMEM((K,), jnp.int32) for _ in range(2)],
            [pltpu.VMEM((K, D), jnp.float32) for _ in range(2)],
            pltpu.VMEM_SHARED((n_rows, D), jnp.float32),
            [pltpu.SemaphoreType.DMA for _ in range(2)],
            [pltpu.SemaphoreType.DMA for _ in range(2)],
            [pltpu.SemaphoreType.DMA for _ in range(2)],
        ],
    )
    def agg(table_hbm, src_hbm, dst_hbm, zeros_hbm, out_hbm,
            sidx, didx, rows, acc, isem, dsem, gsem):
        cid = lax.axis_index("c")
        sid = lax.axis_index("s")

        pltpu.sync_copy(zeros_hbm, rows[0])
        nfull = rpt // K
        for z in range(nfull):
            pltpu.sync_copy(rows[0], acc.at[pl.ds(sid * rpt + z * K, K)])
        if rpt % K:
            pltpu.sync_copy(rows[0].at[pl.ds(0, rpt % K)],
                            acc.at[pl.ds(sid * rpt + nfull * K, rpt % K)])
        plsc.subcore_barrier()

        base = jnp.where(cid == 0, sid * ept[0], e0 + sid * ept[1])
        nch_c = jnp.where(cid == 0, nch[0], nch[1])

        for b in range(2):
            off = base + b * K
            pltpu.async_copy(src_hbm.at[pl.ds(off, K)], sidx[b], isem[b])
            pltpu.async_copy(dst_hbm.at[pl.ds(off, K)], didx[b], dsem[b])

        @pl.loop(0, nch_c // 2)
        def _(r):
            ci = r * 2
            for b in range(2):
                pltpu.make_async_copy(
                    src_hbm.at[pl.ds(base, K)], sidx[b], isem[b]).wait()
            g0 = pltpu.async_copy(table_hbm.at[sidx[0]], rows[0], gsem[0])
            g1 = pltpu.async_copy(table_hbm.at[sidx[1]], rows[1], gsem[1])
            g0.wait()
            g1.wait()

            @pl.when(ci + 2 < nch_c)
            def _():
                pltpu.async_copy(src_hbm.at[pl.ds(base + (ci + 2) * K, K)],
                                 sidx[0], isem[0])
                pltpu.async_copy(src_hbm.at[pl.ds(base + (ci + 3) * K, K)],
                                 sidx[1], isem[1])

            pltpu.make_async_copy(
                dst_hbm.at[pl.ds(base, K)], didx[0], dsem[0]).wait()
            pltpu.sync_copy(rows[0], acc.at[didx[0]], add=True)

            @pl.when(ci + 2 < nch_c)
            def _():
                pltpu.async_copy(dst_hbm.at[pl.ds(base + (ci + 2) * K, K)],
                                 didx[0], dsem[0])

            pltpu.make_async_copy(
                dst_hbm.at[pl.ds(base, K)], didx[1], dsem[1]).wait()
            pltpu.sync_copy(rows[1], acc.at[didx[1]], add=True)

            @pl.when(ci + 2 < nch_c)
            def _():
                pltpu.async_copy(dst_hbm.at[pl.ds(base + (ci + 3) * K, K)],
                                 didx[1], dsem[1])

        plsc.subcore_barrier()
        pltpu.sync_copy(acc.at[pl.ds(sid * rpt, rpt)],
                        out_hbm.at[pl.ds(cid * n_rows + sid * rpt, rpt)])

    return agg


@functools.lru_cache(maxsize=None)
def _make_deg(n_rows, e):
    ept = e // NW
    nfull = ept // K
    rem = ept - nfull * K
    hrows = n_rows // 128
    ctiles = hrows // 8
    mesh = plsc.VectorSubcoreMesh(
        core_axis_name="c", subcore_axis_name="s", num_cores=NC, num_subcores=NS
    )
    cp = pltpu.CompilerParams()
    if "needs_layout_passes" in pltpu.CompilerParams.__dataclass_fields__:
        cp = dataclasses.replace(cp, needs_layout_passes=False)

    @functools.partial(
        pl.kernel,
        out_type=jax.ShapeDtypeStruct((NC * hrows, 128), jnp.float32),
        mesh=mesh,
        compiler_params=cp,
        scratch_types=[
            pltpu.VMEM((K,), jnp.int32),
            pltpu.VMEM((hrows, 128), jnp.float32),
            pltpu.VMEM((hrows,), jnp.int32),
            pltpu.VMEM_SHARED((hrows, 128), jnp.float32),
        ],
    )
    def deg(dst_hbm, iota_hbm, zeros_hbm, out_hbm, didx, hist, iota_v, acc):
        cid = lax.axis_index("c")
        sid = lax.axis_index("s")
        wid = cid * NS + sid

        pltpu.sync_copy(zeros_hbm, hist)

        @pl.when(sid < ctiles)
        def _():
            pltpu.sync_copy(zeros_hbm.at[pl.ds(sid * 8, 8)],
                            acc.at[pl.ds(sid * 8, 8)])

        pltpu.sync_copy(iota_hbm, iota_v)

        base = wid * ept
        ones16 = jnp.full((16,), 1.0, jnp.float32)

        def count16(j):
            v = didx[pl.ds(j * 16, 16)]
            plsc.addupdate_scatter(
                hist,
                [lax.shift_right_logical(v, 7), lax.bitwise_and(v, 127)],
                ones16,
            )

        @pl.loop(0, nfull)
        def _(ci):
            pltpu.sync_copy(dst_hbm.at[pl.ds(base + ci * K, K)], didx)
            for j in range(K // 16):
                count16(j)

        if rem:
            pltpu.sync_copy(dst_hbm.at[pl.ds(base + nfull * K, rem)],
                            didx.at[pl.ds(0, rem)])
            for j in range(rem // 16):
                count16(j)

        plsc.subcore_barrier()
        pltpu.sync_copy(hist, acc.at[iota_v], add=True)
        plsc.subcore_barrier()

        @pl.when(sid < ctiles)
        def _():
            pltpu.sync_copy(acc.at[pl.ds(sid * 8, 8)],
                            out_hbm.at[pl.ds(cid * hrows + sid * 8, 8)])

    return deg


def _prep_body(n, r, x_ref, d0_ref, d1_ref, s0_ref, nb_ref):
    deg = d0_ref[...] + d1_ref[...] + 1.0
    norm = lax.rsqrt(jnp.maximum(deg, 1.0))
    x = x_ref[...]
    rn = jnp.sqrt(jnp.sum(x * x, axis=1, keepdims=True))
    s0_ref[...] = x / jnp.maximum(rn, 1e-12) * norm
    nb_ref[...] = jnp.broadcast_to(norm, x.shape)


def _layer_body(n, r, p0_ref, p1_ref, s_ref, nb_ref, w_ref, b_ref, out_ref):
    nb = nb_ref[...]
    t = (p0_ref[...] + p1_ref[...] + s_ref[...]) * nb
    u = jnp.dot(t, w_ref[...], preferred_element_type=jnp.float32) + b_ref[...]
    sn = jax.nn.relu(u) * nb
    row = pl.program_id(0) * r + lax.broadcasted_iota(jnp.int32, sn.shape, 0)
    out_ref[...] = jnp.where(row < n, sn, 0.0)


def _layer2_body(n, r, p0_ref, p1_ref, s_ref, nb_ref, w_ref, b_ref, w3_ref,
                 g_ref):
    nb = nb_ref[...]
    t = (p0_ref[...] + p1_ref[...] + s_ref[...]) * nb
    u = jnp.dot(t, w_ref[...], preferred_element_type=jnp.float32) + b_ref[...]
    s3 = jax.nn.relu(u) * nb
    row = pl.program_id(0) * r + lax.broadcasted_iota(jnp.int32, s3.shape, 0)
    s3 = jnp.where(row < n, s3, 0.0)
    g_ref[...] = jnp.dot(s3, w3_ref[...], preferred_element_type=jnp.float32)


def _final_body(q0_ref, q1_ref, g_ref, nb_ref, b3_ref, out_ref):
    nb = nb_ref[:, 0:64]
    acc = q0_ref[:, 0:64] + q1_ref[:, 0:64] + g_ref[:, 0:64]
    out_ref[...] = acc * nb + b3_ref[...]


def kernel(x, edge_index, W0, b0, W1, b1, W2, b2, W3, b3):
    n, d_in = x.shape
    d_out = W3.shape[1]
    e = edge_index.shape[1]
    ep = _round_up(e, NW * K)
    np_rows = _round_up(n + 1, 128)
    nd_rows = _round_up(n + 1, 1024)
    hrows = nd_rows // 128
    r = np_rows // 8
    grid = (8,)

    src = edge_index[0]
    dst = edge_index[1]
    pad = ep - e
    srcp = jnp.concatenate([src, jnp.full((pad,), n, jnp.int32)])
    dstp = jnp.concatenate([dst, jnp.zeros((pad,), jnp.int32)])
    zrow = jnp.zeros((K, D), jnp.float32)
    zhist = jnp.zeros((hrows, 128), jnp.float32)
    iota_h = jnp.arange(hrows, dtype=jnp.int32)
    xp = jnp.pad(x, ((0, np_rows - n), (0, 0)))
    w3p = jnp.pad(W3, ((0, 0), (0, d_in - d_out)))

    agg = _make_agg(np_rows, ep)
    degk = _make_deg(nd_rows, e)

    degp = degk(dst, iota_h, zhist)
    d0 = degp[:hrows].reshape(nd_rows, 1)[:np_rows]
    d1 = degp[hrows:].reshape(nd_rows, 1)[:np_rows]

    row_spec = pl.BlockSpec((r, d_in), lambda i: (i, 0))
    row1_spec = pl.BlockSpec((r, 1), lambda i: (i, 0))
    row64_spec = pl.BlockSpec((r, d_out), lambda i: (i, 0))
    w_spec = pl.BlockSpec((d_in, d_in), lambda i: (0, 0))
    b_spec = pl.BlockSpec((1, d_in), lambda i: (0, 0))
    b3_spec = pl.BlockSpec((1, d_out), lambda i: (0, 0))
    fshape = jax.ShapeDtypeStruct((np_rows, d_in), jnp.float32)

    s0, normbc = pl.pallas_call(
        functools.partial(_prep_body, n, r),
        grid=grid,
        in_specs=[row_spec, row1_spec, row1_spec],
        out_specs=[row_spec, row_spec],
        out_shape=[fshape, fshape],
    )(xp, d0, d1)

    layer = pl.pallas_call(
        functools.partial(_layer_body, n, r),
        grid=grid,
        in_specs=[row_spec, row_spec, row_spec, row_spec, w_spec, b_spec],
        out_specs=row_spec,
        out_shape=fshape,
    )
    b0r, b1r, b2r = (bb.reshape(1, -1) for bb in (b0, b1, b2))
    b3r = b3.reshape(1, -1)

    p = agg(s0, srcp, dstp, zrow)
    s1 = layer(p[:np_rows], p[np_rows:], s0, normbc, W0, b0r)
    p = agg(s1, srcp, dstp, zrow)
    s2 = layer(p[:np_rows], p[np_rows:], s1, normbc, W1, b1r)
    p = agg(s2, srcp, dstp, zrow)
    g = pl.pallas_call(
        functools.partial(_layer2_body, n, r),
        grid=grid,
        in_specs=[row_spec, row_spec, row_spec, row_spec, w_spec, b_spec,
                  w_spec],
        out_specs=row_spec,
        out_shape=fshape,
    )(p[:np_rows], p[np_rows:], s2, normbc, W2, b2r, w3p)

    q = agg(g, srcp, dstp, zrow)
    out = pl.pallas_call(
        _final_body,
        grid=grid,
        in_specs=[row_spec, row_spec, row_spec, row_spec, b3_spec],
        out_specs=row64_spec,
        out_shape=jax.ShapeDtypeStruct((np_rows, d_out), jnp.float32),
    )(q[:np_rows], q[np_rows:], g, normbc, b3r)
    return out[:n]


# speedup vs baseline: 1.0628x; 1.0056x over previous
import dataclasses
import functools

import jax
import jax.numpy as jnp
from jax import lax
from jax.experimental import pallas as pl
from jax.experimental.pallas import tpu as pltpu
from jax.experimental.pallas import tpu_sc as plsc

NC = 2
NS = 16
NW = NC * NS
K = 128
ZR = 64
D = 128


def _round_up(v, m):
    return (v + m - 1) // m * m


F0 = 0.68


@functools.lru_cache(maxsize=None)
def _make_agg(n_rows, ep):
    e0 = (int(ep * F0) // (NS * K * 2)) * (NS * K * 2)
    ept = (e0 // NS, (ep - e0) // NS)
    nch = (ept[0] // K, ept[1] // K)
    rpt = n_rows // NS
    mesh = plsc.VectorSubcoreMesh(
        core_axis_name="c", subcore_axis_name="s", num_cores=NC, num_subcores=NS
    )

    @functools.partial(
        pl.kernel,
        out_type=jax.ShapeDtypeStruct((NC * n_rows, D), jnp.float32),
        mesh=mesh,
        scratch_types=[
            [pltpu.VMEM((K,), jnp.int32) for _ in range(2)],
            [pltpu.VMEM((K,), jnp.int32) for _ in range(2)],
            [pltpu.VMEM((K, D), jnp.float32) for _ in range(2)],
            pltpu.VMEM_SHARED((n_rows, D), jnp.float32),
            [pltpu.SemaphoreType.DMA for _ in range(2)],
            [pltpu.SemaphoreType.DMA for _ in range(2)],
            [pltpu.SemaphoreType.DMA for _ in range(2)],
            [pltpu.SemaphoreType.DMA for _ in range(2)],
        ],
    )
    def agg(table_hbm, src_hbm, dst_hbm, zeros_hbm, out_hbm,
            sidx, didx, rows, acc, isem, dsem, gsem, ssem):
        cid = lax.axis_index("c")
        sid = lax.axis_index("s")

        pltpu.sync_copy(zeros_hbm, rows[0])
        nfull = rpt // K
        for z in range(nfull):
            pltpu.sync_copy(rows[0], acc.at[pl.ds(sid * rpt + z * K, K)])
        if rpt % K:
            pltpu.sync_copy(rows[0].at[pl.ds(0, rpt % K)],
                            acc.at[pl.ds(sid * rpt + nfull * K, rpt % K)])
        plsc.subcore_barrier()

        base = jnp.where(cid == 0, sid * ept[0], e0 + sid * ept[1])
        nch_c = jnp.where(cid == 0, nch[0], nch[1])

        for b in range(2):
            off = base + b * K
            pltpu.async_copy(src_hbm.at[pl.ds(off, K)], sidx[b], isem[b])
            pltpu.async_copy(dst_hbm.at[pl.ds(off, K)], didx[b], dsem[b])

        @pl.loop(0, nch_c // 2)
        def _(r):
            ci = r * 2
            for b in range(2):
                pltpu.make_async_copy(
                    src_hbm.at[pl.ds(base, K)], sidx[b], isem[b]).wait()
            g0 = pltpu.async_copy(table_hbm.at[sidx[0]], rows[0], gsem[0])
            g1 = pltpu.async_copy(table_hbm.at[sidx[1]], rows[1], gsem[1])
            g0.wait()
            g1.wait()

            @pl.when(ci + 2 < nch_c)
            def _():
                pltpu.async_copy(src_hbm.at[pl.ds(base + (ci + 2) * K, K)],
                                 sidx[0], isem[0])
                pltpu.async_copy(src_hbm.at[pl.ds(base + (ci + 3) * K, K)],
                                 sidx[1], isem[1])

            for b in range(2):
                pltpu.make_async_copy(
                    dst_hbm.at[pl.ds(base, K)], didx[b], dsem[b]).wait()
            s0 = pltpu.async_copy(rows[0], acc.at[didx[0]], ssem[0], add=True)
            s1 = pltpu.async_copy(rows[1], acc.at[didx[1]], ssem[1], add=True)
            s0.wait()
            s1.wait()

            @pl.when(ci + 2 < nch_c)
            def _():
                pltpu.async_copy(dst_hbm.at[pl.ds(base + (ci + 2) * K, K)],
                                 didx[0], dsem[0])
                pltpu.async_copy(dst_hbm.at[pl.ds(base + (ci + 3) * K, K)],
                                 didx[1], dsem[1])

        plsc.subcore_barrier()
        pltpu.sync_copy(acc.at[pl.ds(sid * rpt, rpt)],
                        out_hbm.at[pl.ds(cid * n_rows + sid * rpt, rpt)])

    return agg


@functools.lru_cache(maxsize=None)
def _make_deg(n_rows, e):
    ept = e // NW
    nfull = ept // K
    rem = ept - nfull * K
    hrows = n_rows // 128
    ctiles = hrows // 8
    mesh = plsc.VectorSubcoreMesh(
        core_axis_name="c", subcore_axis_name="s", num_cores=NC, num_subcores=NS
    )
    cp = pltpu.CompilerParams()
    if "needs_layout_passes" in pltpu.CompilerParams.__dataclass_fields__:
        cp = dataclasses.replace(cp, needs_layout_passes=False)

    @functools.partial(
        pl.kernel,
        out_type=jax.ShapeDtypeStruct((NC * hrows, 128), jnp.float32),
        mesh=mesh,
        compiler_params=cp,
        scratch_types=[
            pltpu.VMEM((K,), jnp.int32),
            pltpu.VMEM((hrows, 128), jnp.float32),
            pltpu.VMEM((hrows,), jnp.int32),
            pltpu.VMEM_SHARED((hrows, 128), jnp.float32),
        ],
    )
    def deg(dst_hbm, iota_hbm, zeros_hbm, out_hbm, didx, hist, iota_v, acc):
        cid = lax.axis_index("c")
        sid = lax.axis_index("s")
        wid = cid * NS + sid

        pltpu.sync_copy(zeros_hbm, hist)

        @pl.when(sid < ctiles)
        def _():
            pltpu.sync_copy(zeros_hbm.at[pl.ds(sid * 8, 8)],
                            acc.at[pl.ds(sid * 8, 8)])

        pltpu.sync_copy(iota_hbm, iota_v)

        base = wid * ept
        ones16 = jnp.full((16,), 1.0, jnp.float32)

        def count16(j):
            v = didx[pl.ds(j * 16, 16)]
            plsc.addupdate_scatter(
                hist,
                [lax.shift_right_logical(v, 7), lax.bitwise_and(v, 127)],
                ones16,
            )

        @pl.loop(0, nfull)
        def _(ci):
            pltpu.sync_copy(dst_hbm.at[pl.ds(base + ci * K, K)], didx)
            for j in range(K // 16):
                count16(j)

        if rem:
            pltpu.sync_copy(dst_hbm.at[pl.ds(base + nfull * K, rem)],
                            didx.at[pl.ds(0, rem)])
            for j in range(rem // 16):
                count16(j)

        plsc.subcore_barrier()
        pltpu.sync_copy(hist, acc.at[iota_v], add=True)
        plsc.subcore_barrier()

        @pl.when(sid < ctiles)
        def _():
            pltpu.sync_copy(acc.at[pl.ds(sid * 8, 8)],
                            out_hbm.at[pl.ds(cid * hrows + sid * 8, 8)])

    return deg


def _prep_body(n, r, x_ref, d0_ref, d1_ref, s0_ref, nb_ref):
    deg = d0_ref[...] + d1_ref[...] + 1.0
    norm = lax.rsqrt(jnp.maximum(deg, 1.0))
    x = x_ref[...]
    rn = jnp.sqrt(jnp.sum(x * x, axis=1, keepdims=True))
    s0_ref[...] = x / jnp.maximum(rn, 1e-12) * norm
    nb_ref[...] = jnp.broadcast_to(norm, x.shape)


def _layer_body(n, r, p0_ref, p1_ref, s_ref, nb_ref, w_ref, b_ref, out_ref):
    nb = nb_ref[...]
    t = (p0_ref[...] + p1_ref[...] + s_ref[...]) * nb
    u = jnp.dot(t, w_ref[...], preferred_element_type=jnp.float32) + b_ref[...]
    sn = jax.nn.relu(u) * nb
    row = pl.program_id(0) * r + lax.broadcasted_iota(jnp.int32, sn.shape, 0)
    out_ref[...] = jnp.where(row < n, sn, 0.0)


def _layer2_body(n, r, p0_ref, p1_ref, s_ref, nb_ref, w_ref, b_ref, w3_ref,
                 g_ref):
    nb = nb_ref[...]
    t = (p0_ref[...] + p1_ref[...] + s_ref[...]) * nb
    u = jnp.dot(t, w_ref[...], preferred_element_type=jnp.float32) + b_ref[...]
    s3 = jax.nn.relu(u) * nb
    row = pl.program_id(0) * r + lax.broadcasted_iota(jnp.int32, s3.shape, 0)
    s3 = jnp.where(row < n, s3, 0.0)
    g_ref[...] = jnp.dot(s3, w3_ref[...], preferred_element_type=jnp.float32)


def _final_body(q0_ref, q1_ref, g_ref, nb_ref, b3_ref, out_ref):
    nb = nb_ref[:, 0:64]
    acc = q0_ref[:, 0:64] + q1_ref[:, 0:64] + g_ref[:, 0:64]
    out_ref[...] = acc * nb + b3_ref[...]


def kernel(x, edge_index, W0, b0, W1, b1, W2, b2, W3, b3):
    n, d_in = x.shape
    d_out = W3.shape[1]
    e = edge_index.shape[1]
    ep = _round_up(e, NW * K)
    np_rows = _round_up(n + 1, 128)
    nd_rows = _round_up(n + 1, 1024)
    hrows = nd_rows // 128
    r = np_rows // 8
    grid = (8,)

    src = edge_index[0]
    dst = edge_index[1]
    pad = ep - e
    srcp = jnp.concatenate([src, jnp.full((pad,), n, jnp.int32)])
    dstp = jnp.concatenate([dst, jnp.zeros((pad,), jnp.int32)])
    zrow = jnp.zeros((K, D), jnp.float32)
    zhist = jnp.zeros((hrows, 128), jnp.float32)
    iota_h = jnp.arange(hrows, dtype=jnp.int32)
    xp = jnp.pad(x, ((0, np_rows - n), (0, 0)))
    w3p = jnp.pad(W3, ((0, 0), (0, d_in - d_out)))

    agg = _make_agg(np_rows, ep)
    degk = _make_deg(nd_rows, e)

    degp = degk(dst, iota_h, zhist)
    d0 = degp[:hrows].reshape(nd_rows, 1)[:np_rows]
    d1 = degp[hrows:].reshape(nd_rows, 1)[:np_rows]

    row_spec = pl.BlockSpec((r, d_in), lambda i: (i, 0))
    row1_spec = pl.BlockSpec((r, 1), lambda i: (i, 0))
    row64_spec = pl.BlockSpec((r, d_out), lambda i: (i, 0))
    w_spec = pl.BlockSpec((d_in, d_in), lambda i: (0, 0))
    b_spec = pl.BlockSpec((1, d_in), lambda i: (0, 0))
    b3_spec = pl.BlockSpec((1, d_out), lambda i: (0, 0))
    fshape = jax.ShapeDtypeStruct((np_rows, d_in), jnp.float32)

    s0, normbc = pl.pallas_call(
        functools.partial(_prep_body, n, r),
        grid=grid,
        in_specs=[row_spec, row1_spec, row1_spec],
        out_specs=[row_spec, row_spec],
        out_shape=[fshape, fshape],
    )(xp, d0, d1)

    layer = pl.pallas_call(
        functools.partial(_layer_body, n, r),
        grid=grid,
        in_specs=[row_spec, row_spec, row_spec, row_spec, w_spec, b_spec],
        out_specs=row_spec,
        out_shape=fshape,
    )
    b0r, b1r, b2r = (bb.reshape(1, -1) for bb in (b0, b1, b2))
    b3r = b3.reshape(1, -1)

    p = agg(s0, srcp, dstp, zrow)
    s1 = layer(p[:np_rows], p[np_rows:], s0, normbc, W0, b0r)
    p = agg(s1, srcp, dstp, zrow)
    s2 = layer(p[:np_rows], p[np_rows:], s1, normbc, W1, b1r)
    p = agg(s2, srcp, dstp, zrow)
    g = pl.pallas_call(
        functools.partial(_layer2_body, n, r),
        grid=grid,
        in_specs=[row_spec, row_spec, row_spec, row_spec, w_spec, b_spec,
                  w_spec],
        out_specs=row_spec,
        out_shape=fshape,
    )(p[:np_rows], p[np_rows:], s2, normbc, W2, b2r, w3p)

    q = agg(g, srcp, dstp, zrow)
    out = pl.pallas_call(
        _final_body,
        grid=grid,
        in_specs=[row_spec, row_spec, row_spec, row_spec, b3_spec],
        out_specs=row64_spec,
        out_shape=jax.ShapeDtypeStruct((np_rows, d_out), jnp.float32),
    )(q[:np_rows], q[np_rows:], g, normbc, b3r)
    return out[:n]


# speedup vs baseline: 1.0630x; 1.0002x over previous
import dataclasses
import functools

import jax
import jax.numpy as jnp
from jax import lax
from jax.experimental import pallas as pl
from jax.experimental.pallas import tpu as pltpu
from jax.experimental.pallas import tpu_sc as plsc

NC = 2
NS = 16
NW = NC * NS
K = 128
D = 128


def _round_up(v, m):
    return (v + m - 1) // m * m


F0 = 0.68


@functools.lru_cache(maxsize=None)
def _make_agg(n_rows, ep):
    e0 = (int(ep * F0) // (NS * K * 2)) * (NS * K * 2)
    ept = (e0 // NS, (ep - e0) // NS)
    nch = (ept[0] // K, ept[1] // K)
    rpt = n_rows // NS
    mesh = plsc.VectorSubcoreMesh(
        core_axis_name="c", subcore_axis_name="s", num_cores=NC, num_subcores=NS
    )

    @functools.partial(
        pl.kernel,
        out_type=jax.ShapeDtypeStruct((NC * n_rows, D), jnp.float32),
        mesh=mesh,
        scratch_types=[
            [pltpu.VMEM((K,), jnp.int32) for _ in range(2)],
            [pltpu.VMEM((K,), jnp.int32) for _ in range(2)],
            [pltpu.VMEM((K, D), jnp.float32) for _ in range(2)],
            pltpu.VMEM_SHARED((n_rows, D), jnp.float32),
            [pltpu.SemaphoreType.DMA for _ in range(2)],
            [pltpu.SemaphoreType.DMA for _ in range(2)],
            [pltpu.SemaphoreType.DMA for _ in range(2)],
            [pltpu.SemaphoreType.DMA for _ in range(2)],
        ],
    )
    def agg(table_hbm, src_hbm, dst_hbm, zeros_hbm, out_hbm,
            sidx, didx, rows, acc, isem, dsem, gsem, ssem):
        cid = lax.axis_index("c")
        sid = lax.axis_index("s")

        pltpu.sync_copy(zeros_hbm, rows[0])
        nfull = rpt // K
        for z in range(nfull):
            pltpu.sync_copy(rows[0], acc.at[pl.ds(sid * rpt + z * K, K)])
        if rpt % K:
            pltpu.sync_copy(rows[0].at[pl.ds(0, rpt % K)],
                            acc.at[pl.ds(sid * rpt + nfull * K, rpt % K)])
        plsc.subcore_barrier()

        base = jnp.where(cid == 0, sid * ept[0], e0 + sid * ept[1])
        nch_c = jnp.where(cid == 0, nch[0], nch[1])

        for b in range(2):
            off = base + b * K
            pltpu.async_copy(src_hbm.at[pl.ds(off, K)], sidx[b], isem[b])
            pltpu.async_copy(dst_hbm.at[pl.ds(off, K)], didx[b], dsem[b])

        @pl.loop(0, nch_c // 2)
        def _(r):
            ci = r * 2
            for b in range(2):
                pltpu.make_async_copy(
                    src_hbm.at[pl.ds(base, K)], sidx[b], isem[b]).wait()
            g0 = pltpu.async_copy(table_hbm.at[sidx[0]], rows[0], gsem[0])
            g1 = pltpu.async_copy(table_hbm.at[sidx[1]], rows[1], gsem[1])
            g0.wait()
            g1.wait()

            @pl.when(ci + 2 < nch_c)
            def _():
                pltpu.async_copy(src_hbm.at[pl.ds(base + (ci + 2) * K, K)],
                                 sidx[0], isem[0])
                pltpu.async_copy(src_hbm.at[pl.ds(base + (ci + 3) * K, K)],
                                 sidx[1], isem[1])

            for b in range(2):
                pltpu.make_async_copy(
                    dst_hbm.at[pl.ds(base, K)], didx[b], dsem[b]).wait()
            s0 = pltpu.async_copy(rows[0], acc.at[didx[0]], ssem[0], add=True)
            s1 = pltpu.async_copy(rows[1], acc.at[didx[1]], ssem[1], add=True)
            s0.wait()
            s1.wait()

            @pl.when(ci + 2 < nch_c)
            def _():
                pltpu.async_copy(dst_hbm.at[pl.ds(base + (ci + 2) * K, K)],
                                 didx[0], dsem[0])
                pltpu.async_copy(dst_hbm.at[pl.ds(base + (ci + 3) * K, K)],
                                 didx[1], dsem[1])

        plsc.subcore_barrier()
        pltpu.sync_copy(acc.at[pl.ds(sid * rpt, rpt)],
                        out_hbm.at[pl.ds(cid * n_rows + sid * rpt, rpt)])

    return agg


@functools.lru_cache(maxsize=None)
def _make_deg(n_rows, e):
    ept = e // NW
    nfull = ept // K
    rem = ept - nfull * K
    hrows = n_rows // 128
    ctiles = hrows // 8
    mesh = plsc.VectorSubcoreMesh(
        core_axis_name="c", subcore_axis_name="s", num_cores=NC, num_subcores=NS
    )
    cp = pltpu.CompilerParams()
    if "needs_layout_passes" in pltpu.CompilerParams.__dataclass_fields__:
        cp = dataclasses.replace(cp, needs_layout_passes=False)

    @functools.partial(
        pl.kernel,
        out_type=jax.ShapeDtypeStruct((NC * hrows, 128), jnp.float32),
        mesh=mesh,
        compiler_params=cp,
        scratch_types=[
            pltpu.VMEM((K,), jnp.int32),
            pltpu.VMEM((hrows, 128), jnp.float32),
            pltpu.VMEM((hrows,), jnp.int32),
            pltpu.VMEM_SHARED((hrows, 128), jnp.float32),
        ],
    )
    def deg(dst_hbm, iota_hbm, zeros_hbm, out_hbm, didx, hist, iota_v, acc):
        cid = lax.axis_index("c")
        sid = lax.axis_index("s")
        wid = cid * NS + sid

        pltpu.sync_copy(zeros_hbm, hist)

        @pl.when(sid < ctiles)
        def _():
            pltpu.sync_copy(zeros_hbm.at[pl.ds(sid * 8, 8)],
                            acc.at[pl.ds(sid * 8, 8)])

        pltpu.sync_copy(iota_hbm, iota_v)

        base = wid * ept
        ones16 = jnp.full((16,), 1.0, jnp.float32)

        def count16(j):
            v = didx[pl.ds(j * 16, 16)]
            plsc.addupdate_scatter(
                hist,
                [lax.shift_right_logical(v, 7), lax.bitwise_and(v, 127)],
                ones16,
            )

        @pl.loop(0, nfull)
        def _(ci):
            pltpu.sync_copy(dst_hbm.at[pl.ds(base + ci * K, K)], didx)
            for j in range(K // 16):
                count16(j)

        if rem:
            pltpu.sync_copy(dst_hbm.at[pl.ds(base + nfull * K, rem)],
                            didx.at[pl.ds(0, rem)])
            for j in range(rem // 16):
                count16(j)

        plsc.subcore_barrier()
        pltpu.sync_copy(hist, acc.at[iota_v], add=True)
        plsc.subcore_barrier()

        @pl.when(sid < ctiles)
        def _():
            pltpu.sync_copy(acc.at[pl.ds(sid * 8, 8)],
                            out_hbm.at[pl.ds(cid * hrows + sid * 8, 8)])

    return deg


def _prep_body(n, r, x_ref, d0_ref, d1_ref, s0_ref, nb_ref):
    deg = d0_ref[...] + d1_ref[...] + 1.0
    norm = lax.rsqrt(jnp.maximum(deg, 1.0))
    x = x_ref[...]
    rn = jnp.sqrt(jnp.sum(x * x, axis=1, keepdims=True))
    s0_ref[...] = x / jnp.maximum(rn, 1e-12) * norm
    nb_ref[...] = jnp.broadcast_to(norm, x.shape)


def _layer_body(n, r, p0_ref, p1_ref, s_ref, nb_ref, w_ref, b_ref, out_ref):
    nb = nb_ref[...]
    t = (p0_ref[...] + p1_ref[...] + s_ref[...]) * nb
    u = jnp.dot(t, w_ref[...], preferred_element_type=jnp.float32) + b_ref[...]
    sn = jax.nn.relu(u) * nb
    row = pl.program_id(0) * r + lax.broadcasted_iota(jnp.int32, sn.shape, 0)
    out_ref[...] = jnp.where(row < n, sn, 0.0)


def _layer2_body(n, r, p0_ref, p1_ref, s_ref, nb_ref, w_ref, b_ref, w3_ref,
                 g_ref):
    nb = nb_ref[...]
    t = (p0_ref[...] + p1_ref[...] + s_ref[...]) * nb
    u = jnp.dot(t, w_ref[...], preferred_element_type=jnp.float32) + b_ref[...]
    s3 = jax.nn.relu(u) * nb
    row = pl.program_id(0) * r + lax.broadcasted_iota(jnp.int32, s3.shape, 0)
    s3 = jnp.where(row < n, s3, 0.0)
    g_ref[...] = jnp.dot(s3, w3_ref[...], preferred_element_type=jnp.float32)


def _final_body(q0_ref, q1_ref, g_ref, nb_ref, b3_ref, out_ref):
    nb = nb_ref[:, 0:64]
    acc = q0_ref[:, 0:64] + q1_ref[:, 0:64] + g_ref[:, 0:64]
    out_ref[...] = acc * nb + b3_ref[...]


def kernel(x, edge_index, W0, b0, W1, b1, W2, b2, W3, b3):
    n, d_in = x.shape
    d_out = W3.shape[1]
    e = edge_index.shape[1]
    ep = _round_up(e, NW * K)
    np_rows = _round_up(n + 1, 128)
    nd_rows = _round_up(n + 1, 1024)
    hrows = nd_rows // 128
    r = np_rows // 8
    grid = (8,)

    src = edge_index[0]
    dst = edge_index[1]
    pad = ep - e
    srcp = jnp.concatenate([src, jnp.full((pad,), n, jnp.int32)])
    dstp = jnp.concatenate([dst, jnp.zeros((pad,), jnp.int32)])
    zrow = jnp.zeros((K, D), jnp.float32)
    zhist = jnp.zeros((hrows, 128), jnp.float32)
    iota_h = jnp.arange(hrows, dtype=jnp.int32)
    xp = jnp.pad(x, ((0, np_rows - n), (0, 0)))
    w3p = jnp.pad(W3, ((0, 0), (0, d_in - d_out)))

    agg = _make_agg(np_rows, ep)
    degk = _make_deg(nd_rows, e)

    degp = degk(dst, iota_h, zhist)
    d0 = degp[:hrows].reshape(nd_rows, 1)[:np_rows]
    d1 = degp[hrows:].reshape(nd_rows, 1)[:np_rows]

    row_spec = pl.BlockSpec((r, d_in), lambda i: (i, 0))
    row1_spec = pl.BlockSpec((r, 1), lambda i: (i, 0))
    row64_spec = pl.BlockSpec((r, d_out), lambda i: (i, 0))
    w_spec = pl.BlockSpec((d_in, d_in), lambda i: (0, 0))
    b_spec = pl.BlockSpec((1, d_in), lambda i: (0, 0))
    b3_spec = pl.BlockSpec((1, d_out), lambda i: (0, 0))
    fshape = jax.ShapeDtypeStruct((np_rows, d_in), jnp.float32)

    s0, normbc = pl.pallas_call(
        functools.partial(_prep_body, n, r),
        grid=grid,
        in_specs=[row_spec, row1_spec, row1_spec],
        out_specs=[row_spec, row_spec],
        out_shape=[fshape, fshape],
    )(xp, d0, d1)

    layer = pl.pallas_call(
        functools.partial(_layer_body, n, r),
        grid=grid,
        in_specs=[row_spec, row_spec, row_spec, row_spec, w_spec, b_spec],
        out_specs=row_spec,
        out_shape=fshape,
    )
    b0r, b1r, b2r = (bb.reshape(1, -1) for bb in (b0, b1, b2))
    b3r = b3.reshape(1, -1)

    p = agg(s0, srcp, dstp, zrow)
    s1 = layer(p[:np_rows], p[np_rows:], s0, normbc, W0, b0r)
    p = agg(s1, srcp, dstp, zrow)
    s2 = layer(p[:np_rows], p[np_rows:], s1, normbc, W1, b1r)
    p = agg(s2, srcp, dstp, zrow)
    g = pl.pallas_call(
        functools.partial(_layer2_body, n, r),
        grid=grid,
        in_specs=[row_spec, row_spec, row_spec, row_spec, w_spec, b_spec,
                  w_spec],
        out_specs=row_spec,
        out_shape=fshape,
    )(p[:np_rows], p[np_rows:], s2, normbc, W2, b2r, w3p)

    q = agg(g, srcp, dstp, zrow)
    out = pl.pallas_call(
        _final_body,
        grid=grid,
        in_specs=[row_spec, row_spec, row_spec, row_spec, b3_spec],
        out_specs=row64_spec,
        out_shape=jax.ShapeDtypeStruct((np_rows, d_out), jnp.float32),
    )(q[:np_rows], q[np_rows:], g, normbc, b3r)
    return out[:n]
